# Initial kernel scaffold; baseline (speedup 1.0000x reference)
import jax
import jax.numpy as jnp
from jax.experimental import pallas as pl


def kernel(x, edge_index, Wl1, Wr1, b1, Wl2, Wr2, b2):
    raise NotImplementedError("write your pallas kernel here")


# speedup vs baseline: 7.8708x; 7.8708x over previous
import functools

import jax
import jax.numpy as jnp
from jax import lax
from jax.experimental import pallas as pl
from jax.experimental.pallas import tpu as pltpu
from jax.experimental.pallas import tpu_sc as plsc

NC = 2
NS = 16
NW = NC * NS
CH = 125


def _sc_aggregate(q, src_r, dst_r, ones_h, npad, d, nchunk, with_deg):
  mesh = plsc.VectorSubcoreMesh(core_axis_name="c", subcore_axis_name="s",
                                num_cores=NC, num_subcores=NS)
  rows_per_tile = npad // NS

  out_type = [jax.ShapeDtypeStruct((NC, npad, d), jnp.float32)]
  scratch = [
      pltpu.VMEM((nchunk, CH), jnp.int32),
      pltpu.VMEM((nchunk, CH), jnp.int32),
      pltpu.VMEM((CH, d), jnp.float32),
      pltpu.VMEM((16, d), jnp.float32),
      pltpu.VMEM_SHARED((npad, d), jnp.float32),
      pltpu.SemaphoreType.DMA,
  ]
  if with_deg:
    out_type.append(jax.ShapeDtypeStruct((NC, npad), jnp.float32))
    scratch += [
        pltpu.VMEM((CH,), jnp.float32),
        pltpu.VMEM((rows_per_tile,), jnp.float32),
        pltpu.VMEM_SHARED((npad,), jnp.float32),
    ]

  @functools.partial(
      pl.kernel, mesh=mesh, out_type=tuple(out_type),
      scratch_types=tuple(scratch))
  def agg_kernel(q_hbm, src_hbm, dst_hbm, ones_hbm, *rest):
    if with_deg:
      (part_hbm, degp_hbm, src_v, dst_v, rows_v, zmat_v, acc_s, sem,
       ones_v, zrow_v, dega_s) = rest
    else:
      (part_hbm, src_v, dst_v, rows_v, zmat_v, acc_s, sem) = rest
    c = lax.axis_index("c")
    s = lax.axis_index("s")
    wid = s * NC + c

    pltpu.sync_copy(src_hbm.at[wid], src_v)
    pltpu.sync_copy(dst_hbm.at[wid], dst_v)

    z16 = jnp.zeros((16,), jnp.float32)
    for i in range(16):
      for j in range(d // 16):
        zmat_v[i, pl.ds(j * 16, 16)] = z16
    base = s * rows_per_tile

    def clear_body(k, _):
      pltpu.sync_copy(zmat_v, acc_s.at[pl.ds(base + k * 16, 16)])
      return 0
    lax.fori_loop(0, rows_per_tile // 16, clear_body, 0)

    if with_deg:
      pltpu.sync_copy(ones_hbm, ones_v)
      for j in range(rows_per_tile // 16):
        zrow_v[pl.ds(j * 16, 16)] = z16
      pltpu.sync_copy(zrow_v, dega_s.at[pl.ds(base, rows_per_tile)])

    plsc.subcore_barrier()

    def chunk_body(j, _):
      pltpu.async_copy(q_hbm.at[src_v.at[j]], rows_v, sem).wait()
      pltpu.sync_copy(rows_v, acc_s.at[dst_v.at[j]], add=True)
      if with_deg:
        pltpu.sync_copy(ones_v, dega_s.at[dst_v.at[j]], add=True)
      return 0
    lax.fori_loop(0, nchunk, chunk_body, 0)

    plsc.subcore_barrier()

    pltpu.sync_copy(acc_s.at[pl.ds(base, rows_per_tile)],
                    part_hbm.at[c, pl.ds(base, rows_per_tile)])
    if with_deg:
      pltpu.sync_copy(dega_s.at[pl.ds(base, rows_per_tile)],
                      degp_hbm.at[c, pl.ds(base, rows_per_tile)])

  return agg_kernel(q, src_r, dst_r, ones_h)


def _tc_transform(x, Wl, Wr):
  n, _ = x.shape
  dout = Wl.shape[1]

  def body(x_ref, wl_ref, wr_ref, q_ref, r_ref):
    xv = x_ref[...]
    q_ref[...] = jnp.dot(xv, wl_ref[...], preferred_element_type=jnp.float32)
    r_ref[...] = jnp.dot(xv, wr_ref[...], preferred_element_type=jnp.float32)

  return pl.pallas_call(
      body,
      out_shape=(jax.ShapeDtypeStruct((n, dout), jnp.float32),
                 jax.ShapeDtypeStruct((n, dout), jnp.float32)),
  )(x, Wl, Wr)


def _tc_mid(part, deg2, r1, b1):
  n, d = r1.shape

  def body(p_ref, d_ref, r1_ref, b1_ref, h_ref):
    deg = jnp.maximum(d_ref[0] + d_ref[1], 1.0)
    h = (p_ref[0] + p_ref[1]) / deg + b1_ref[...] + r1_ref[...]
    h_ref[...] = jnp.maximum(h, 0.0)

  return pl.pallas_call(
      body,
      out_shape=jax.ShapeDtypeStruct((n, d), jnp.float32),
  )(part, deg2, r1, b1.reshape(1, -1))


def _tc_final(part, deg2, h, b2, Wl2, Wr2):
  n = h.shape[0]
  dout = Wl2.shape[1]

  def body(p_ref, d_ref, h_ref, b2_ref, wl_ref, wr_ref, o_ref):
    deg = jnp.maximum(d_ref[0] + d_ref[1], 1.0)
    mean2 = (p_ref[0] + p_ref[1]) / deg
    o = (jnp.dot(mean2, wl_ref[...], preferred_element_type=jnp.float32)
         + b2_ref[...]
         + jnp.dot(h_ref[...], wr_ref[...],
                   preferred_element_type=jnp.float32))
    m = jnp.max(o, axis=-1, keepdims=True)
    e = jnp.exp(o - m)
    lse = jnp.log(jnp.sum(e, axis=-1, keepdims=True)) + m
    o_ref[...] = o - lse

  return pl.pallas_call(
      body,
      out_shape=jax.ShapeDtypeStruct((n, dout), jnp.float32),
  )(part, deg2, h, b2.reshape(1, -1), Wl2, Wr2)


def kernel(x, edge_index, Wl1, Wr1, b1, Wl2, Wr2, b2):
  n, d_in = x.shape
  e = edge_index.shape[1]
  assert e % (NW * CH) == 0
  nchunk = e // (NW * CH)
  npad = ((n + NW * 16 - 1) // (NW * 16)) * (NW * 16)

  xp = jnp.pad(x, ((0, npad - n), (0, 0)))
  src_r = edge_index[0].reshape(NW, nchunk, CH)
  dst_r = edge_index[1].reshape(NW, nchunk, CH)
  ones_h = jnp.ones((CH,), jnp.float32)

  q1, r1 = _tc_transform(xp, Wl1, Wr1)
  part1, degp = _sc_aggregate(q1, src_r, dst_r, ones_h, npad, d_in,
                              nchunk, with_deg=True)
  deg2 = degp.reshape(NC, npad, 1)
  h = _tc_mid(part1, deg2, r1, b1)

  (part2,) = _sc_aggregate(h, src_r, dst_r, ones_h, npad, d_in,
                           nchunk, with_deg=False)
  out = _tc_final(part2, deg2, h, b2, Wl2, Wr2)
  return out[:n]
